# initial kernel scaffold (unmeasured)
import functools

import jax
import jax.numpy as jnp
from jax import lax
from jax.experimental import pallas as pl
from jax.experimental.pallas import tpu as pltpu

N_DEV = 32
SQ = 256
D = 1024
SKV = 4096
DH = 128
H_LOCAL = 8
KV_LOCAL = 2
SCALE = 0.08838834764831843


def _compute_body(x_ref, wq_ref, wo_ref, k_hbm, v_hbm,
                  out_ref, k_vmem, v_vmem, cp_sems):
    my_i = lax.axis_index("i")
    kv0 = my_i * KV_LOCAL

    k_cp = pltpu.make_async_copy(
        k_hbm.at[0, :, pl.ds(kv0, KV_LOCAL), :], k_vmem, cp_sems.at[0])
    v_cp = pltpu.make_async_copy(
        v_hbm.at[0, :, pl.ds(kv0, KV_LOCAL), :], v_vmem, cp_sems.at[1])
    k_cp.start()
    v_cp.start()

    q = jnp.dot(x_ref[0], wq_ref[...], preferred_element_type=jnp.float32)

    k_cp.wait()
    v_cp.wait()

    outs = []
    for h in range(H_LOCAL):
        q_h = q[:, h * DH:(h + 1) * DH]
        k_h = k_vmem[:, h // 4, :]
        v_h = v_vmem[:, h // 4, :]
        s = lax.dot_general(
            q_h, k_h, (((1,), (1,)), ((), ())),
            preferred_element_type=jnp.float32) * SCALE
        m = jnp.max(s, axis=1, keepdims=True)
        p = jnp.exp(s - m)
        l = jnp.sum(p, axis=1, keepdims=True)
        o_h = jnp.dot(p, v_h, preferred_element_type=jnp.float32) / l
        outs.append(o_h)
    attn = jnp.concatenate(outs, axis=1)

    out_ref[...] = jnp.dot(attn, wo_ref[...], preferred_element_type=jnp.float32)


def _compute_partial(x, Wq, Wo, K_ext, V_ext):
    return pl.pallas_call(
        _compute_body,
        out_shape=jax.ShapeDtypeStruct((SQ, D), jnp.float32),
        in_specs=[
            pl.BlockSpec(memory_space=pltpu.VMEM),
            pl.BlockSpec(memory_space=pltpu.VMEM),
            pl.BlockSpec(memory_space=pltpu.VMEM),
            pl.BlockSpec(memory_space=pltpu.ANY),
            pl.BlockSpec(memory_space=pltpu.ANY),
        ],
        out_specs=pl.BlockSpec(memory_space=pltpu.VMEM),
        scratch_shapes=[
            pltpu.VMEM((SKV, KV_LOCAL, DH), jnp.float32),
            pltpu.VMEM((SKV, KV_LOCAL, DH), jnp.float32),
            pltpu.SemaphoreType.DMA((2,)),
        ],
    )(x, Wq, Wo, K_ext, V_ext)


def _allreduce_body(partial_ref, out_ref, comm_ref,
                    send_sems, recv_sems, credit_sem):
    my_i = lax.axis_index("i")
    left = (my_i - 1) % N_DEV
    right = (my_i + 1) % N_DEV

    barrier_sem = pltpu.get_barrier_semaphore()
    for nbr in (left, right):
        pl.semaphore_signal(barrier_sem, inc=1, device_id=(nbr,),
                            device_id_type=pl.DeviceIdType.MESH)
    pl.semaphore_wait(barrier_sem, 2)

    out_ref[0, :, :] = partial_ref[...]

    for h in range(N_DEV - 1):
        slot = h % 2
        if h >= 2:
            pl.semaphore_wait(credit_sem, 1)
        src = partial_ref if h == 0 else comm_ref.at[(h - 1) % 2]
        rdma = pltpu.make_async_remote_copy(
            src_ref=src,
            dst_ref=comm_ref.at[slot],
            send_sem=send_sems.at[slot],
            recv_sem=recv_sems.at[slot],
            device_id=(right,),
            device_id_type=pl.DeviceIdType.MESH,
        )
        rdma.start()
        rdma.wait_send()
        if 1 <= h <= N_DEV - 3:
            pl.semaphore_signal(credit_sem, inc=1, device_id=(left,),
                                device_id_type=pl.DeviceIdType.MESH)
        rdma.wait_recv()
        out_ref[0, :, :] = out_ref[0, :, :] + comm_ref[slot, :, :]


def _ring_allreduce(partial):
    return pl.pallas_call(
        _allreduce_body,
        out_shape=jax.ShapeDtypeStruct((1, SQ, D), jnp.float32),
        in_specs=[pl.BlockSpec(memory_space=pltpu.VMEM)],
        out_specs=pl.BlockSpec(memory_space=pltpu.VMEM),
        scratch_shapes=[
            pltpu.VMEM((2, SQ, D), jnp.float32),
            pltpu.SemaphoreType.DMA((2,)),
            pltpu.SemaphoreType.DMA((2,)),
            pltpu.SemaphoreType.REGULAR,
        ],
        compiler_params=pltpu.CompilerParams(collective_id=0),
    )(partial)


def kernel(x, Wq, Wo, K_ext, V_ext):
    partial = _compute_partial(x, Wq, Wo, K_ext, V_ext)
    return _ring_allreduce(partial)


# baseline (device time: 502443 ns/iter reference)
import functools

import jax
import jax.numpy as jnp
from jax import lax
from jax.experimental import pallas as pl
from jax.experimental.pallas import tpu as pltpu

N_DEV = 32
SQ = 256
D = 1024
SKV = 4096
DH = 128
H_LOCAL = 8
KV_LOCAL = 2
SCALE = 0.08838834764831843


def _compute_body(x_ref, wq_ref, wo_ref, k_hbm, v_hbm,
                  out_ref, k_vmem, v_vmem, cp_sems):
    my_i = lax.axis_index("i")
    kv0 = my_i * KV_LOCAL

    k_cp = pltpu.make_async_copy(
        k_hbm.at[0, :, pl.ds(kv0, KV_LOCAL), :], k_vmem, cp_sems.at[0])
    v_cp = pltpu.make_async_copy(
        v_hbm.at[0, :, pl.ds(kv0, KV_LOCAL), :], v_vmem, cp_sems.at[1])
    k_cp.start()
    v_cp.start()

    q = jnp.dot(x_ref[0], wq_ref[...], preferred_element_type=jnp.float32)

    k_cp.wait()
    v_cp.wait()

    outs = []
    for h in range(H_LOCAL):
        q_h = q[:, h * DH:(h + 1) * DH]
        k_h = k_vmem[:, h // 4, :]
        v_h = v_vmem[:, h // 4, :]
        s = lax.dot_general(
            q_h, k_h, (((1,), (1,)), ((), ())),
            preferred_element_type=jnp.float32) * SCALE
        m = jnp.max(s, axis=1, keepdims=True)
        p = jnp.exp(s - m)
        l = jnp.sum(p, axis=1, keepdims=True)
        o_h = jnp.dot(p, v_h, preferred_element_type=jnp.float32) / l
        outs.append(o_h)
    attn = jnp.concatenate(outs, axis=1)

    out_ref[...] = jnp.dot(attn, wo_ref[...], preferred_element_type=jnp.float32)


def _compute_partial(x, Wq, Wo, K_ext, V_ext):
    return pl.pallas_call(
        _compute_body,
        out_shape=jax.ShapeDtypeStruct((SQ, D), jnp.float32),
        in_specs=[
            pl.BlockSpec(memory_space=pltpu.VMEM),
            pl.BlockSpec(memory_space=pltpu.VMEM),
            pl.BlockSpec(memory_space=pltpu.VMEM),
            pl.BlockSpec(memory_space=pl.ANY),
            pl.BlockSpec(memory_space=pl.ANY),
        ],
        out_specs=pl.BlockSpec(memory_space=pltpu.VMEM),
        scratch_shapes=[
            pltpu.VMEM((SKV, KV_LOCAL, DH), jnp.float32),
            pltpu.VMEM((SKV, KV_LOCAL, DH), jnp.float32),
            pltpu.SemaphoreType.DMA((2,)),
        ],
        compiler_params=pltpu.CompilerParams(
            vmem_limit_bytes=100 * 1024 * 1024),
    )(x, Wq, Wo, K_ext, V_ext)


def _allreduce_body(partial_ref, out_ref, comm_ref,
                    send_sems, recv_sems, credit_sem):
    my_i = lax.axis_index("i")
    left = (my_i - 1) % N_DEV
    right = (my_i + 1) % N_DEV

    barrier_sem = pltpu.get_barrier_semaphore()
    for nbr in (left, right):
        pl.semaphore_signal(barrier_sem, inc=1, device_id=(nbr,),
                            device_id_type=pl.DeviceIdType.MESH)
    pl.semaphore_wait(barrier_sem, 2)

    out_ref[0, :, :] = partial_ref[...]

    for h in range(N_DEV - 1):
        slot = h % 2
        if h >= 2:
            pl.semaphore_wait(credit_sem, 1)
        src = partial_ref if h == 0 else comm_ref.at[(h - 1) % 2]
        rdma = pltpu.make_async_remote_copy(
            src_ref=src,
            dst_ref=comm_ref.at[slot],
            send_sem=send_sems.at[slot],
            recv_sem=recv_sems.at[slot],
            device_id=(right,),
            device_id_type=pl.DeviceIdType.MESH,
        )
        rdma.start()
        rdma.wait_send()
        if 1 <= h <= N_DEV - 3:
            pl.semaphore_signal(credit_sem, inc=1, device_id=(left,),
                                device_id_type=pl.DeviceIdType.MESH)
        rdma.wait_recv()
        out_ref[0, :, :] = out_ref[0, :, :] + comm_ref[slot, :, :]


def _ring_allreduce(partial):
    return pl.pallas_call(
        _allreduce_body,
        out_shape=jax.ShapeDtypeStruct((1, SQ, D), jnp.float32),
        in_specs=[pl.BlockSpec(memory_space=pltpu.VMEM)],
        out_specs=pl.BlockSpec(memory_space=pltpu.VMEM),
        scratch_shapes=[
            pltpu.VMEM((2, SQ, D), jnp.float32),
            pltpu.SemaphoreType.DMA((2,)),
            pltpu.SemaphoreType.DMA((2,)),
            pltpu.SemaphoreType.REGULAR,
        ],
        compiler_params=pltpu.CompilerParams(collective_id=0),
    )(partial)


def kernel(x, Wq, Wo, K_ext, V_ext):
    partial = _compute_partial(x, Wq, Wo, K_ext, V_ext)
    return _ring_allreduce(partial)


# device time: 74716 ns/iter; 6.7247x vs baseline; 6.7247x over previous
import functools

import jax
import jax.numpy as jnp
from jax import lax
from jax.experimental import pallas as pl
from jax.experimental.pallas import tpu as pltpu

N_DEV = 32
SQ = 256
D = 1024
SKV = 4096
DH = 128
H_LOCAL = 8
KV_LOCAL = 2
SCALE = 0.08838834764831843


def _compute_body(x_ref, wq_ref, wo_ref, k_hbm, v_hbm,
                  out_ref, k_vmem, v_vmem, cp_sems):
    my_i = lax.axis_index("i")
    kv0 = my_i * KV_LOCAL

    k_cp = pltpu.make_async_copy(
        k_hbm.at[0, :, pl.ds(kv0, KV_LOCAL), :], k_vmem, cp_sems.at[0])
    v_cp = pltpu.make_async_copy(
        v_hbm.at[0, :, pl.ds(kv0, KV_LOCAL), :], v_vmem, cp_sems.at[1])
    k_cp.start()
    v_cp.start()

    q = jnp.dot(x_ref[0], wq_ref[...], preferred_element_type=jnp.float32)

    k_cp.wait()
    v_cp.wait()

    outs = []
    for h in range(H_LOCAL):
        q_h = q[:, h * DH:(h + 1) * DH]
        k_h = k_vmem[:, h // 4, :]
        v_h = v_vmem[:, h // 4, :]
        s = lax.dot_general(
            q_h, k_h, (((1,), (1,)), ((), ())),
            preferred_element_type=jnp.float32) * SCALE
        m = jnp.max(s, axis=1, keepdims=True)
        p = jnp.exp(s - m)
        l = jnp.sum(p, axis=1, keepdims=True)
        o_h = jnp.dot(p, v_h, preferred_element_type=jnp.float32) / l
        outs.append(o_h)
    attn = jnp.concatenate(outs, axis=1)

    out_ref[...] = jnp.dot(attn, wo_ref[...], preferred_element_type=jnp.float32)


def _compute_partial(x, Wq, Wo, K_ext, V_ext):
    return pl.pallas_call(
        _compute_body,
        out_shape=jax.ShapeDtypeStruct((SQ, D), jnp.float32),
        in_specs=[
            pl.BlockSpec(memory_space=pltpu.VMEM),
            pl.BlockSpec(memory_space=pltpu.VMEM),
            pl.BlockSpec(memory_space=pltpu.VMEM),
            pl.BlockSpec(memory_space=pl.ANY),
            pl.BlockSpec(memory_space=pl.ANY),
        ],
        out_specs=pl.BlockSpec(memory_space=pltpu.VMEM),
        scratch_shapes=[
            pltpu.VMEM((SKV, KV_LOCAL, DH), jnp.float32),
            pltpu.VMEM((SKV, KV_LOCAL, DH), jnp.float32),
            pltpu.SemaphoreType.DMA((2,)),
        ],
        compiler_params=pltpu.CompilerParams(
            vmem_limit_bytes=100 * 1024 * 1024),
    )(x, Wq, Wo, K_ext, V_ext)


def _allreduce_body(partial_ref, out_ref, comm_ref, send_sems, recv_sems):
    m = lax.axis_index("i")
    z = m // 8
    y = (m % 8) // 2
    x = (m % 2) ^ (y & 1)

    def midx(px, py, pz):
        return pz * 8 + py * 2 + (px ^ (py & 1))

    pidx = [
        midx(1 - x, y, z),
        midx(x, y ^ 1, z),
        midx(x, y ^ 2, z),
        midx(x, y, z ^ 1),
        midx(x, y, z ^ 2),
    ]
    bits = [x, y & 1, (y >> 1) & 1, z & 1, (z >> 1) & 1]

    barrier_sem = pltpu.get_barrier_semaphore()
    for k in range(5):
        pl.semaphore_signal(barrier_sem, inc=1, device_id=(pidx[k],),
                            device_id_type=pl.DeviceIdType.MESH)
    pl.semaphore_wait(barrier_sem, 5)

    out_ref[0, :, :] = partial_ref[...]

    off = m * 0
    for s in range(5):
        half = 128 >> s
        b = bits[s]
        send_off = pl.multiple_of(off + (1 - b) * half, 8)
        rdma = pltpu.make_async_remote_copy(
            src_ref=out_ref.at[0, pl.ds(send_off, half), :],
            dst_ref=comm_ref.at[s, 0:half, :],
            send_sem=send_sems.at[s],
            recv_sem=recv_sems.at[s],
            device_id=(pidx[s],),
            device_id_type=pl.DeviceIdType.MESH,
        )
        rdma.start()
        rdma.wait_send()
        rdma.wait_recv()
        off = pl.multiple_of(off + b * half, 8)
        out_ref[0, pl.ds(off, half), :] = (
            out_ref[0, pl.ds(off, half), :] + comm_ref[s, 0:half, :])

    for s in range(5, 10):
        q = 9 - s
        lsz = 8 << (s - 5)
        aoff = pl.multiple_of(off, 8)
        rdma = pltpu.make_async_remote_copy(
            src_ref=out_ref.at[0, pl.ds(aoff, lsz), :],
            dst_ref=out_ref.at[0, pl.ds(aoff, lsz), :],
            send_sem=send_sems.at[s],
            recv_sem=recv_sems.at[s],
            device_id=(pidx[q],),
            device_id_type=pl.DeviceIdType.MESH,
        )
        rdma.start()
        rdma.wait_send()
        rdma.wait_recv()
        off = off & ~lsz


def _ring_allreduce(partial):
    return pl.pallas_call(
        _allreduce_body,
        out_shape=jax.ShapeDtypeStruct((1, SQ, D), jnp.float32),
        in_specs=[pl.BlockSpec(memory_space=pltpu.VMEM)],
        out_specs=pl.BlockSpec(memory_space=pltpu.VMEM),
        scratch_shapes=[
            pltpu.VMEM((5, 128, D), jnp.float32),
            pltpu.SemaphoreType.DMA((10,)),
            pltpu.SemaphoreType.DMA((10,)),
        ],
        compiler_params=pltpu.CompilerParams(collective_id=0),
    )(partial)


def kernel(x, Wq, Wo, K_ext, V_ext):
    partial = _compute_partial(x, Wq, Wo, K_ext, V_ext)
    return _ring_allreduce(partial)


# device time: 73412 ns/iter; 6.8442x vs baseline; 1.0178x over previous
import jax
import jax.numpy as jnp
from jax import lax
from jax.experimental import pallas as pl
from jax.experimental.pallas import tpu as pltpu

N_DEV = 32
SQ = 256
D = 1024
SKV = 4096
DH = 128
H_LOCAL = 8
KV_LOCAL = 2
SCALE = 0.08838834764831843

MESH = pl.DeviceIdType.MESH


def _compute_body(x_ref, wq_ref, wo_ref, k_hbm, v_hbm,
                  out_ref, k_vmem, v_vmem, cp_sems):
    my_i = lax.axis_index("i")
    kv0 = my_i * KV_LOCAL

    k_cp = pltpu.make_async_copy(
        k_hbm.at[0, :, pl.ds(kv0, KV_LOCAL), :], k_vmem, cp_sems.at[0])
    v_cp = pltpu.make_async_copy(
        v_hbm.at[0, :, pl.ds(kv0, KV_LOCAL), :], v_vmem, cp_sems.at[1])
    k_cp.start()
    v_cp.start()

    xb = x_ref[0].astype(jnp.bfloat16)
    wqb = wq_ref[...].astype(jnp.bfloat16)
    q = jnp.dot(xb, wqb, preferred_element_type=jnp.float32)

    k_cp.wait()
    v_cp.wait()
    kb = k_vmem[...].astype(jnp.bfloat16)
    vb = v_vmem[...].astype(jnp.bfloat16)

    outs = []
    for h in range(H_LOCAL):
        q_h = q[:, h * DH:(h + 1) * DH].astype(jnp.bfloat16)
        k_h = kb[:, h // 4, :]
        v_h = vb[:, h // 4, :]
        s = lax.dot_general(
            q_h, k_h, (((1,), (1,)), ((), ())),
            preferred_element_type=jnp.float32) * SCALE
        m = jnp.max(s, axis=1, keepdims=True)
        p = jnp.exp(s - m)
        l = jnp.sum(p, axis=1, keepdims=True)
        o_h = jnp.dot(p.astype(jnp.bfloat16), v_h,
                      preferred_element_type=jnp.float32) / l
        outs.append(o_h)
    attn = jnp.concatenate(outs, axis=1).astype(jnp.bfloat16)

    wob = wo_ref[...].astype(jnp.bfloat16)
    out_ref[...] = jnp.dot(attn, wob, preferred_element_type=jnp.float32)


def _compute_partial(x, Wq, Wo, K_ext, V_ext):
    return pl.pallas_call(
        _compute_body,
        out_shape=jax.ShapeDtypeStruct((SQ, D), jnp.float32),
        in_specs=[
            pl.BlockSpec(memory_space=pltpu.VMEM),
            pl.BlockSpec(memory_space=pltpu.VMEM),
            pl.BlockSpec(memory_space=pltpu.VMEM),
            pl.BlockSpec(memory_space=pl.ANY),
            pl.BlockSpec(memory_space=pl.ANY),
        ],
        out_specs=pl.BlockSpec(memory_space=pltpu.VMEM),
        scratch_shapes=[
            pltpu.VMEM((SKV, KV_LOCAL, DH), jnp.float32),
            pltpu.VMEM((SKV, KV_LOCAL, DH), jnp.float32),
            pltpu.SemaphoreType.DMA((2,)),
        ],
        compiler_params=pltpu.CompilerParams(
            vmem_limit_bytes=100 * 1024 * 1024),
    )(x, Wq, Wo, K_ext, V_ext)


_COMM_X = 0
_COMM_Y = 128
_COMM_Z = 224
_SEM_RS_X = 0
_SEM_RS_Y = 1
_SEM_RS_Z = 4
_SEM_AG_Z = 7
_SEM_AG_Y = 10
_SEM_AG_X = 13
N_SEMS = 14


def _allreduce_body(partial_ref, out_ref, comm_ref, send_sems, recv_sems):
    m = lax.axis_index("i")
    z = m // 8
    y = (m % 8) // 2
    x = (m % 2) ^ (y & 1)

    def midx(px, py, pz):
        return pz * 8 + py * 2 + (px ^ (py & 1))

    x_partner = midx(1 - x, y, z)
    y_partner = [midx(x, (y + d) % 4, z) for d in (1, 2, 3)]
    z_partner = [midx(x, y, (z + d) % 4) for d in (1, 2, 3)]

    barrier_sem = pltpu.get_barrier_semaphore()
    for p in [x_partner] + y_partner + z_partner:
        pl.semaphore_signal(barrier_sem, inc=1, device_id=(p,),
                            device_id_type=MESH)
    pl.semaphore_wait(barrier_sem, 7)

    out_ref[0, :, :] = partial_ref[...]

    xoff = pl.multiple_of(x * 128, 8)
    rdma = pltpu.make_async_remote_copy(
        src_ref=out_ref.at[0, pl.ds(pl.multiple_of((1 - x) * 128, 8), 128), :],
        dst_ref=comm_ref.at[_COMM_X:_COMM_X + 128, :],
        send_sem=send_sems.at[_SEM_RS_X],
        recv_sem=recv_sems.at[_SEM_RS_X],
        device_id=(x_partner,),
        device_id_type=MESH,
    )
    rdma.start()
    rdma.wait_send()
    rdma.wait_recv()
    out_ref[0, pl.ds(xoff, 128), :] = (
        out_ref[0, pl.ds(xoff, 128), :] + comm_ref[_COMM_X:_COMM_X + 128, :])

    rs_y = []
    for d in (1, 2, 3):
        p = (y + d) % 4
        soff = pl.multiple_of(xoff + 32 * p, 8)
        r = pltpu.make_async_remote_copy(
            src_ref=out_ref.at[0, pl.ds(soff, 32), :],
            dst_ref=comm_ref.at[_COMM_Y + 32 * (d - 1):_COMM_Y + 32 * d, :],
            send_sem=send_sems.at[_SEM_RS_Y + d - 1],
            recv_sem=recv_sems.at[_SEM_RS_Y + d - 1],
            device_id=(y_partner[d - 1],),
            device_id_type=MESH,
        )
        r.start()
        rs_y.append(r)
    yoff = pl.multiple_of(xoff + 32 * y, 8)
    for r in rs_y:
        r.wait_send()
    for r in rs_y:
        r.wait_recv()
    out_ref[0, pl.ds(yoff, 32), :] = (
        out_ref[0, pl.ds(yoff, 32), :]
        + comm_ref[_COMM_Y:_COMM_Y + 32, :]
        + comm_ref[_COMM_Y + 32:_COMM_Y + 64, :]
        + comm_ref[_COMM_Y + 64:_COMM_Y + 96, :])

    rs_z = []
    for d in (1, 2, 3):
        p = (z + d) % 4
        soff = pl.multiple_of(yoff + 8 * p, 8)
        r = pltpu.make_async_remote_copy(
            src_ref=out_ref.at[0, pl.ds(soff, 8), :],
            dst_ref=comm_ref.at[_COMM_Z + 8 * (d - 1):_COMM_Z + 8 * d, :],
            send_sem=send_sems.at[_SEM_RS_Z + d - 1],
            recv_sem=recv_sems.at[_SEM_RS_Z + d - 1],
            device_id=(z_partner[d - 1],),
            device_id_type=MESH,
        )
        r.start()
        rs_z.append(r)
    zoff = pl.multiple_of(yoff + 8 * z, 8)
    for r in rs_z:
        r.wait_send()
    for r in rs_z:
        r.wait_recv()
    out_ref[0, pl.ds(zoff, 8), :] = (
        out_ref[0, pl.ds(zoff, 8), :]
        + comm_ref[_COMM_Z:_COMM_Z + 8, :]
        + comm_ref[_COMM_Z + 8:_COMM_Z + 16, :]
        + comm_ref[_COMM_Z + 16:_COMM_Z + 24, :])

    ag_z = []
    for d in (1, 2, 3):
        r = pltpu.make_async_remote_copy(
            src_ref=out_ref.at[0, pl.ds(zoff, 8), :],
            dst_ref=out_ref.at[0, pl.ds(zoff, 8), :],
            send_sem=send_sems.at[_SEM_AG_Z + d - 1],
            recv_sem=recv_sems.at[_SEM_AG_Z + d - 1],
            device_id=(z_partner[d - 1],),
            device_id_type=MESH,
        )
        r.start()
        ag_z.append(r)
    for r in ag_z:
        r.wait_send()
    for r in ag_z:
        r.wait_recv()

    ag_y = []
    for d in (1, 2, 3):
        r = pltpu.make_async_remote_copy(
            src_ref=out_ref.at[0, pl.ds(yoff, 32), :],
            dst_ref=out_ref.at[0, pl.ds(yoff, 32), :],
            send_sem=send_sems.at[_SEM_AG_Y + d - 1],
            recv_sem=recv_sems.at[_SEM_AG_Y + d - 1],
            device_id=(y_partner[d - 1],),
            device_id_type=MESH,
        )
        r.start()
        ag_y.append(r)
    for r in ag_y:
        r.wait_send()
    for r in ag_y:
        r.wait_recv()

    rdma = pltpu.make_async_remote_copy(
        src_ref=out_ref.at[0, pl.ds(xoff, 128), :],
        dst_ref=out_ref.at[0, pl.ds(xoff, 128), :],
        send_sem=send_sems.at[_SEM_AG_X],
        recv_sem=recv_sems.at[_SEM_AG_X],
        device_id=(x_partner,),
        device_id_type=MESH,
    )
    rdma.start()
    rdma.wait_send()
    rdma.wait_recv()


def _allreduce(partial):
    return pl.pallas_call(
        _allreduce_body,
        out_shape=jax.ShapeDtypeStruct((1, SQ, D), jnp.float32),
        in_specs=[pl.BlockSpec(memory_space=pltpu.VMEM)],
        out_specs=pl.BlockSpec(memory_space=pltpu.VMEM),
        scratch_shapes=[
            pltpu.VMEM((248, D), jnp.float32),
            pltpu.SemaphoreType.DMA((N_SEMS,)),
            pltpu.SemaphoreType.DMA((N_SEMS,)),
        ],
        compiler_params=pltpu.CompilerParams(collective_id=0),
    )(partial)


def kernel(x, Wq, Wo, K_ext, V_ext):
    partial = _compute_partial(x, Wq, Wo, K_ext, V_ext)
    return _allreduce(partial)


# device time: 58747 ns/iter; 8.5527x vs baseline; 1.2496x over previous
import jax
import jax.numpy as jnp
from jax import lax
from jax.experimental import pallas as pl
from jax.experimental.pallas import tpu as pltpu

N_DEV = 32
SQ = 256
D = 1024
SKV = 4096
DH = 128
H_LOCAL = 8
KV_LOCAL = 2
SCALE = 0.08838834764831843

MESH = pl.DeviceIdType.MESH


def _compute_body(x_ref, wq_ref, wo_ref, k_hbm, v_hbm,
                  out_ref, k_vmem, v_vmem, cp_sems):
    my_i = lax.axis_index("i")
    kv0 = my_i * KV_LOCAL

    k_cp = pltpu.make_async_copy(
        k_hbm.at[0, :, pl.ds(kv0, KV_LOCAL), :], k_vmem, cp_sems.at[0])
    v_cp = pltpu.make_async_copy(
        v_hbm.at[0, :, pl.ds(kv0, KV_LOCAL), :], v_vmem, cp_sems.at[1])
    k_cp.start()
    v_cp.start()

    q = jnp.dot(x_ref[0], wq_ref[...], preferred_element_type=jnp.float32)

    k_cp.wait()
    v_cp.wait()

    outs = []
    for h in range(H_LOCAL):
        q_h = q[:, h * DH:(h + 1) * DH]
        k_h = k_vmem[:, h // 4, :]
        v_h = v_vmem[:, h // 4, :]
        s = lax.dot_general(
            q_h, k_h, (((1,), (1,)), ((), ())),
            preferred_element_type=jnp.float32) * SCALE
        m = jnp.max(s, axis=1, keepdims=True)
        p = jnp.exp(s - m)
        l = jnp.sum(p, axis=1, keepdims=True)
        o_h = jnp.dot(p, v_h, preferred_element_type=jnp.float32) / l
        outs.append(o_h)
    attn = jnp.concatenate(outs, axis=1)

    out_ref[...] = jnp.dot(attn, wo_ref[...], preferred_element_type=jnp.float32)


def _compute_partial(x, Wq, Wo, K_ext, V_ext):
    return pl.pallas_call(
        _compute_body,
        out_shape=jax.ShapeDtypeStruct((SQ, D), jnp.float32),
        in_specs=[
            pl.BlockSpec(memory_space=pltpu.VMEM),
            pl.BlockSpec(memory_space=pltpu.VMEM),
            pl.BlockSpec(memory_space=pltpu.VMEM),
            pl.BlockSpec(memory_space=pl.ANY),
            pl.BlockSpec(memory_space=pl.ANY),
        ],
        out_specs=pl.BlockSpec(memory_space=pltpu.VMEM),
        scratch_shapes=[
            pltpu.VMEM((SKV, KV_LOCAL, DH), jnp.float32),
            pltpu.VMEM((SKV, KV_LOCAL, DH), jnp.float32),
            pltpu.SemaphoreType.DMA((2,)),
        ],
        compiler_params=pltpu.CompilerParams(
            vmem_limit_bytes=100 * 1024 * 1024),
    )(x, Wq, Wo, K_ext, V_ext)


_CR_X = 0
_CR_Y = 128
_CR_Z = 224
_CA_Z = 248
_CA_Y = 272
_CA_X = 368
_SEM_RS_X = 0
_SEM_RS_Y = 1
_SEM_RS_Z = 4
_SEM_AG_Z = 7
_SEM_AG_Y = 10
_SEM_AG_X = 13
N_SEMS = 14


def _allreduce_body(partial_ref, out_ref, comm_ref, stage_ref,
                    send_sems, recv_sems):
    m = lax.axis_index("i")
    z = m // 8
    y = (m % 8) // 2
    x = (m % 2) ^ (y & 1)

    def midx(px, py, pz):
        return pz * 8 + py * 2 + (px ^ (py & 1))

    x_partner = midx(1 - x, y, z)
    y_partner = [midx(x, (y + d) % 4, z) for d in (1, 2, 3)]
    z_partner = [midx(x, y, (z + d) % 4) for d in (1, 2, 3)]

    barrier_sem = pltpu.get_barrier_semaphore()
    for p in [x_partner] + y_partner + z_partner:
        pl.semaphore_signal(barrier_sem, inc=1, device_id=(p,),
                            device_id_type=MESH)
    pl.semaphore_wait(barrier_sem, 7)

    out_ref[0, :, :] = partial_ref[...]
    f32 = jnp.float32
    bf16 = jnp.bfloat16

    xoff = pl.multiple_of(x * 128, 8)
    send_off = pl.multiple_of((1 - x) * 128, 8)
    stage_ref[0:128, :] = out_ref[0, pl.ds(send_off, 128), :].astype(bf16)
    rdma = pltpu.make_async_remote_copy(
        src_ref=stage_ref.at[0:128, :],
        dst_ref=comm_ref.at[_CR_X:_CR_X + 128, :],
        send_sem=send_sems.at[_SEM_RS_X],
        recv_sem=recv_sems.at[_SEM_RS_X],
        device_id=(x_partner,),
        device_id_type=MESH,
    )
    rdma.start()
    rdma.wait_send()
    rdma.wait_recv()
    out_ref[0, pl.ds(xoff, 128), :] = (
        out_ref[0, pl.ds(xoff, 128), :]
        + comm_ref[_CR_X:_CR_X + 128, :].astype(f32))

    rs_y = []
    for d in (1, 2, 3):
        p = (y + d) % 4
        soff = pl.multiple_of(xoff + 32 * p, 8)
        stage_ref[32 * (d - 1):32 * d, :] = (
            out_ref[0, pl.ds(soff, 32), :].astype(bf16))
        r = pltpu.make_async_remote_copy(
            src_ref=stage_ref.at[32 * (d - 1):32 * d, :],
            dst_ref=comm_ref.at[_CR_Y + 32 * (d - 1):_CR_Y + 32 * d, :],
            send_sem=send_sems.at[_SEM_RS_Y + d - 1],
            recv_sem=recv_sems.at[_SEM_RS_Y + d - 1],
            device_id=(y_partner[d - 1],),
            device_id_type=MESH,
        )
        r.start()
        rs_y.append(r)
    yoff = pl.multiple_of(xoff + 32 * y, 8)
    for r in rs_y:
        r.wait_send()
    for r in rs_y:
        r.wait_recv()
    out_ref[0, pl.ds(yoff, 32), :] = (
        out_ref[0, pl.ds(yoff, 32), :]
        + comm_ref[_CR_Y:_CR_Y + 32, :].astype(f32)
        + comm_ref[_CR_Y + 32:_CR_Y + 64, :].astype(f32)
        + comm_ref[_CR_Y + 64:_CR_Y + 96, :].astype(f32))

    rs_z = []
    for d in (1, 2, 3):
        p = (z + d) % 4
        soff = pl.multiple_of(yoff + 8 * p, 8)
        stage_ref[96 + 8 * (d - 1):96 + 8 * d, :] = (
            out_ref[0, pl.ds(soff, 8), :].astype(bf16))
        r = pltpu.make_async_remote_copy(
            src_ref=stage_ref.at[96 + 8 * (d - 1):96 + 8 * d, :],
            dst_ref=comm_ref.at[_CR_Z + 8 * (d - 1):_CR_Z + 8 * d, :],
            send_sem=send_sems.at[_SEM_RS_Z + d - 1],
            recv_sem=recv_sems.at[_SEM_RS_Z + d - 1],
            device_id=(z_partner[d - 1],),
            device_id_type=MESH,
        )
        r.start()
        rs_z.append(r)
    zoff = pl.multiple_of(yoff + 8 * z, 8)
    for r in rs_z:
        r.wait_send()
    for r in rs_z:
        r.wait_recv()
    out_ref[0, pl.ds(zoff, 8), :] = (
        out_ref[0, pl.ds(zoff, 8), :]
        + comm_ref[_CR_Z:_CR_Z + 8, :].astype(f32)
        + comm_ref[_CR_Z + 8:_CR_Z + 16, :].astype(f32)
        + comm_ref[_CR_Z + 16:_CR_Z + 24, :].astype(f32))

    stage_ref[0:8, :] = out_ref[0, pl.ds(zoff, 8), :].astype(bf16)
    ag_z = []
    for d in (1, 2, 3):
        r = pltpu.make_async_remote_copy(
            src_ref=stage_ref.at[0:8, :],
            dst_ref=comm_ref.at[_CA_Z + 8 * (d - 1):_CA_Z + 8 * d, :],
            send_sem=send_sems.at[_SEM_AG_Z + d - 1],
            recv_sem=recv_sems.at[_SEM_AG_Z + d - 1],
            device_id=(z_partner[d - 1],),
            device_id_type=MESH,
        )
        r.start()
        ag_z.append(r)
    for r in ag_z:
        r.wait_send()
    for r in ag_z:
        r.wait_recv()
    for d in (1, 2, 3):
        src_z = (z - d) % 4
        roff = pl.multiple_of(yoff + 8 * src_z, 8)
        out_ref[0, pl.ds(roff, 8), :] = (
            comm_ref[_CA_Z + 8 * (d - 1):_CA_Z + 8 * d, :].astype(f32))

    stage_ref[0:32, :] = out_ref[0, pl.ds(yoff, 32), :].astype(bf16)
    ag_y = []
    for d in (1, 2, 3):
        r = pltpu.make_async_remote_copy(
            src_ref=stage_ref.at[0:32, :],
            dst_ref=comm_ref.at[_CA_Y + 32 * (d - 1):_CA_Y + 32 * d, :],
            send_sem=send_sems.at[_SEM_AG_Y + d - 1],
            recv_sem=recv_sems.at[_SEM_AG_Y + d - 1],
            device_id=(y_partner[d - 1],),
            device_id_type=MESH,
        )
        r.start()
        ag_y.append(r)
    for r in ag_y:
        r.wait_send()
    for r in ag_y:
        r.wait_recv()
    for d in (1, 2, 3):
        src_y = (y - d) % 4
        roff = pl.multiple_of(xoff + 32 * src_y, 8)
        out_ref[0, pl.ds(roff, 32), :] = (
            comm_ref[_CA_Y + 32 * (d - 1):_CA_Y + 32 * d, :].astype(f32))

    stage_ref[0:128, :] = out_ref[0, pl.ds(xoff, 128), :].astype(bf16)
    rdma = pltpu.make_async_remote_copy(
        src_ref=stage_ref.at[0:128, :],
        dst_ref=comm_ref.at[_CA_X:_CA_X + 128, :],
        send_sem=send_sems.at[_SEM_AG_X],
        recv_sem=recv_sems.at[_SEM_AG_X],
        device_id=(x_partner,),
        device_id_type=MESH,
    )
    rdma.start()
    rdma.wait_send()
    rdma.wait_recv()
    out_ref[0, pl.ds(send_off, 128), :] = (
        comm_ref[_CA_X:_CA_X + 128, :].astype(f32))


def _allreduce(partial):
    return pl.pallas_call(
        _allreduce_body,
        out_shape=jax.ShapeDtypeStruct((1, SQ, D), jnp.float32),
        in_specs=[pl.BlockSpec(memory_space=pltpu.VMEM)],
        out_specs=pl.BlockSpec(memory_space=pltpu.VMEM),
        scratch_shapes=[
            pltpu.VMEM((496, D), jnp.bfloat16),
            pltpu.VMEM((128, D), jnp.bfloat16),
            pltpu.SemaphoreType.DMA((N_SEMS,)),
            pltpu.SemaphoreType.DMA((N_SEMS,)),
        ],
        compiler_params=pltpu.CompilerParams(collective_id=0),
    )(partial)


def kernel(x, Wq, Wo, K_ext, V_ext):
    partial = _compute_partial(x, Wq, Wo, K_ext, V_ext)
    return _allreduce(partial)


# device time: 58582 ns/iter; 8.5767x vs baseline; 1.0028x over previous
import jax
import jax.numpy as jnp
from jax import lax
from jax.experimental import pallas as pl
from jax.experimental.pallas import tpu as pltpu

N_DEV = 32
SQ = 256
D = 1024
SKV = 4096
DH = 128
H_LOCAL = 8
KV_LOCAL = 2
SCALE = 0.08838834764831843

MESH = pl.DeviceIdType.MESH

_CR_X = 0
_CR_Y = 128
_CR_Z = 224
_CA_Z = 248
_CA_Y = 272
_CA_X = 368
_SEM_RS_X = 0
_SEM_RS_Y = 1
_SEM_RS_Z = 4
_SEM_AG_Z = 7
_SEM_AG_Y = 10
_SEM_AG_X = 13
N_SEMS = 14


def _body(x_ref, wq_ref, wo_ref, k_hbm, v_hbm, out_ref,
          k_vmem, v_vmem, cp_sems, comm_ref, stage_ref,
          send_sems, recv_sems):
    m = lax.axis_index("i")
    z = m // 8
    y = (m % 8) // 2
    x = (m % 2) ^ (y & 1)

    def midx(px, py, pz):
        return pz * 8 + py * 2 + (px ^ (py & 1))

    x_partner = midx(1 - x, y, z)
    y_partner = [midx(x, (y + d) % 4, z) for d in (1, 2, 3)]
    z_partner = [midx(x, y, (z + d) % 4) for d in (1, 2, 3)]

    barrier_sem = pltpu.get_barrier_semaphore()
    for p in [x_partner] + y_partner + z_partner:
        pl.semaphore_signal(barrier_sem, inc=1, device_id=(p,),
                            device_id_type=MESH)
    pl.semaphore_wait(barrier_sem, 7)

    kv0 = m * KV_LOCAL
    k_cp = pltpu.make_async_copy(
        k_hbm.at[0, :, pl.ds(kv0, KV_LOCAL), :], k_vmem, cp_sems.at[0])
    v_cp = pltpu.make_async_copy(
        v_hbm.at[0, :, pl.ds(kv0, KV_LOCAL), :], v_vmem, cp_sems.at[1])
    k_cp.start()
    v_cp.start()

    q = jnp.dot(x_ref[0], wq_ref[...], preferred_element_type=jnp.float32)

    k_cp.wait()
    v_cp.wait()

    outs = []
    for h in range(H_LOCAL):
        q_h = q[:, h * DH:(h + 1) * DH]
        k_h = k_vmem[:, h // 4, :]
        v_h = v_vmem[:, h // 4, :]
        s = lax.dot_general(
            q_h, k_h, (((1,), (1,)), ((), ())),
            preferred_element_type=jnp.float32) * SCALE
        mx = jnp.max(s, axis=1, keepdims=True)
        p = jnp.exp(s - mx)
        l = jnp.sum(p, axis=1, keepdims=True)
        o_h = jnp.dot(p, v_h, preferred_element_type=jnp.float32) / l
        outs.append(o_h)
    attn = jnp.concatenate(outs, axis=1)

    out_ref[0, :, :] = jnp.dot(attn, wo_ref[...],
                               preferred_element_type=jnp.float32)

    f32 = jnp.float32
    bf16 = jnp.bfloat16

    xoff = pl.multiple_of(x * 128, 8)
    send_off = pl.multiple_of((1 - x) * 128, 8)
    stage_ref[0:128, :] = out_ref[0, pl.ds(send_off, 128), :].astype(bf16)
    rdma = pltpu.make_async_remote_copy(
        src_ref=stage_ref.at[0:128, :],
        dst_ref=comm_ref.at[_CR_X:_CR_X + 128, :],
        send_sem=send_sems.at[_SEM_RS_X],
        recv_sem=recv_sems.at[_SEM_RS_X],
        device_id=(x_partner,),
        device_id_type=MESH,
    )
    rdma.start()
    rdma.wait_send()
    rdma.wait_recv()
    out_ref[0, pl.ds(xoff, 128), :] = (
        out_ref[0, pl.ds(xoff, 128), :]
        + comm_ref[_CR_X:_CR_X + 128, :].astype(f32))

    rs_y = []
    for d in (1, 2, 3):
        p = (y + d) % 4
        soff = pl.multiple_of(xoff + 32 * p, 8)
        stage_ref[32 * (d - 1):32 * d, :] = (
            out_ref[0, pl.ds(soff, 32), :].astype(bf16))
        r = pltpu.make_async_remote_copy(
            src_ref=stage_ref.at[32 * (d - 1):32 * d, :],
            dst_ref=comm_ref.at[_CR_Y + 32 * (d - 1):_CR_Y + 32 * d, :],
            send_sem=send_sems.at[_SEM_RS_Y + d - 1],
            recv_sem=recv_sems.at[_SEM_RS_Y + d - 1],
            device_id=(y_partner[d - 1],),
            device_id_type=MESH,
        )
        r.start()
        rs_y.append(r)
    yoff = pl.multiple_of(xoff + 32 * y, 8)
    for r in rs_y:
        r.wait_send()
    for r in rs_y:
        r.wait_recv()
    out_ref[0, pl.ds(yoff, 32), :] = (
        out_ref[0, pl.ds(yoff, 32), :]
        + comm_ref[_CR_Y:_CR_Y + 32, :].astype(f32)
        + comm_ref[_CR_Y + 32:_CR_Y + 64, :].astype(f32)
        + comm_ref[_CR_Y + 64:_CR_Y + 96, :].astype(f32))

    rs_z = []
    for d in (1, 2, 3):
        p = (z + d) % 4
        soff = pl.multiple_of(yoff + 8 * p, 8)
        stage_ref[96 + 8 * (d - 1):96 + 8 * d, :] = (
            out_ref[0, pl.ds(soff, 8), :].astype(bf16))
        r = pltpu.make_async_remote_copy(
            src_ref=stage_ref.at[96 + 8 * (d - 1):96 + 8 * d, :],
            dst_ref=comm_ref.at[_CR_Z + 8 * (d - 1):_CR_Z + 8 * d, :],
            send_sem=send_sems.at[_SEM_RS_Z + d - 1],
            recv_sem=recv_sems.at[_SEM_RS_Z + d - 1],
            device_id=(z_partner[d - 1],),
            device_id_type=MESH,
        )
        r.start()
        rs_z.append(r)
    zoff = pl.multiple_of(yoff + 8 * z, 8)
    for r in rs_z:
        r.wait_send()
    for r in rs_z:
        r.wait_recv()
    out_ref[0, pl.ds(zoff, 8), :] = (
        out_ref[0, pl.ds(zoff, 8), :]
        + comm_ref[_CR_Z:_CR_Z + 8, :].astype(f32)
        + comm_ref[_CR_Z + 8:_CR_Z + 16, :].astype(f32)
        + comm_ref[_CR_Z + 16:_CR_Z + 24, :].astype(f32))

    stage_ref[0:8, :] = out_ref[0, pl.ds(zoff, 8), :].astype(bf16)
    ag_z = []
    for d in (1, 2, 3):
        r = pltpu.make_async_remote_copy(
            src_ref=stage_ref.at[0:8, :],
            dst_ref=comm_ref.at[_CA_Z + 8 * (d - 1):_CA_Z + 8 * d, :],
            send_sem=send_sems.at[_SEM_AG_Z + d - 1],
            recv_sem=recv_sems.at[_SEM_AG_Z + d - 1],
            device_id=(z_partner[d - 1],),
            device_id_type=MESH,
        )
        r.start()
        ag_z.append(r)
    for r in ag_z:
        r.wait_send()
    for r in ag_z:
        r.wait_recv()
    for d in (1, 2, 3):
        src_z = (z - d) % 4
        roff = pl.multiple_of(yoff + 8 * src_z, 8)
        out_ref[0, pl.ds(roff, 8), :] = (
            comm_ref[_CA_Z + 8 * (d - 1):_CA_Z + 8 * d, :].astype(f32))

    stage_ref[0:32, :] = out_ref[0, pl.ds(yoff, 32), :].astype(bf16)
    ag_y = []
    for d in (1, 2, 3):
        r = pltpu.make_async_remote_copy(
            src_ref=stage_ref.at[0:32, :],
            dst_ref=comm_ref.at[_CA_Y + 32 * (d - 1):_CA_Y + 32 * d, :],
            send_sem=send_sems.at[_SEM_AG_Y + d - 1],
            recv_sem=recv_sems.at[_SEM_AG_Y + d - 1],
            device_id=(y_partner[d - 1],),
            device_id_type=MESH,
        )
        r.start()
        ag_y.append(r)
    for r in ag_y:
        r.wait_send()
    for r in ag_y:
        r.wait_recv()
    for d in (1, 2, 3):
        src_y = (y - d) % 4
        roff = pl.multiple_of(xoff + 32 * src_y, 8)
        out_ref[0, pl.ds(roff, 32), :] = (
            comm_ref[_CA_Y + 32 * (d - 1):_CA_Y + 32 * d, :].astype(f32))

    stage_ref[0:128, :] = out_ref[0, pl.ds(xoff, 128), :].astype(bf16)
    rdma = pltpu.make_async_remote_copy(
        src_ref=stage_ref.at[0:128, :],
        dst_ref=comm_ref.at[_CA_X:_CA_X + 128, :],
        send_sem=send_sems.at[_SEM_AG_X],
        recv_sem=recv_sems.at[_SEM_AG_X],
        device_id=(x_partner,),
        device_id_type=MESH,
    )
    rdma.start()
    rdma.wait_send()
    rdma.wait_recv()
    out_ref[0, pl.ds(send_off, 128), :] = (
        comm_ref[_CA_X:_CA_X + 128, :].astype(f32))


def kernel(x, Wq, Wo, K_ext, V_ext):
    return pl.pallas_call(
        _body,
        out_shape=jax.ShapeDtypeStruct((1, SQ, D), jnp.float32),
        in_specs=[
            pl.BlockSpec(memory_space=pltpu.VMEM),
            pl.BlockSpec(memory_space=pltpu.VMEM),
            pl.BlockSpec(memory_space=pltpu.VMEM),
            pl.BlockSpec(memory_space=pl.ANY),
            pl.BlockSpec(memory_space=pl.ANY),
        ],
        out_specs=pl.BlockSpec(memory_space=pltpu.VMEM),
        scratch_shapes=[
            pltpu.VMEM((SKV, KV_LOCAL, DH), jnp.float32),
            pltpu.VMEM((SKV, KV_LOCAL, DH), jnp.float32),
            pltpu.SemaphoreType.DMA((2,)),
            pltpu.VMEM((496, D), jnp.bfloat16),
            pltpu.VMEM((128, D), jnp.bfloat16),
            pltpu.SemaphoreType.DMA((N_SEMS,)),
            pltpu.SemaphoreType.DMA((N_SEMS,)),
        ],
        compiler_params=pltpu.CompilerParams(
            collective_id=0,
            vmem_limit_bytes=100 * 1024 * 1024),
    )(x, Wq, Wo, K_ext, V_ext)


# device time: 53393 ns/iter; 9.4103x vs baseline; 1.0972x over previous
import jax
import jax.numpy as jnp
from jax import lax
from jax.experimental import pallas as pl
from jax.experimental.pallas import tpu as pltpu

N_DEV = 32
SQ = 256
D = 1024
SKV = 4096
DH = 128
H_LOCAL = 8
KV_LOCAL = 2
SCALE = 0.08838834764831843

MESH = pl.DeviceIdType.MESH

_CR = 0
_CA = 256
_SEM_RS = 0
_SEM_AG = 31
N_SEMS = 62


def _body(x_ref, wq_ref, wo_ref, k_hbm, v_hbm, out_ref,
          k_vmem, v_vmem, cp_sems, comm_ref, stage_ref,
          send_sems, recv_sems):
    m = lax.axis_index("i")

    barrier_sem = pltpu.get_barrier_semaphore()
    for d in range(1, N_DEV):
        pl.semaphore_signal(barrier_sem, inc=1, device_id=((m + d) % N_DEV,),
                            device_id_type=MESH)
    pl.semaphore_wait(barrier_sem, N_DEV - 1)

    kv0 = m * KV_LOCAL
    k_cp = pltpu.make_async_copy(
        k_hbm.at[0, :, pl.ds(kv0, KV_LOCAL), :], k_vmem, cp_sems.at[0])
    v_cp = pltpu.make_async_copy(
        v_hbm.at[0, :, pl.ds(kv0, KV_LOCAL), :], v_vmem, cp_sems.at[1])
    k_cp.start()
    v_cp.start()

    q = jnp.dot(x_ref[0], wq_ref[...], preferred_element_type=jnp.float32)

    k_cp.wait()
    v_cp.wait()

    outs = []
    for h in range(H_LOCAL):
        q_h = q[:, h * DH:(h + 1) * DH]
        k_h = k_vmem[:, h // 4, :]
        v_h = v_vmem[:, h // 4, :]
        s = lax.dot_general(
            q_h, k_h, (((1,), (1,)), ((), ())),
            preferred_element_type=jnp.float32) * SCALE
        mx = jnp.max(s, axis=1, keepdims=True)
        p = jnp.exp(s - mx)
        l = jnp.sum(p, axis=1, keepdims=True)
        o_h = jnp.dot(p, v_h, preferred_element_type=jnp.float32) / l
        outs.append(o_h)
    attn = jnp.concatenate(outs, axis=1)

    out_ref[0, :, :] = jnp.dot(attn, wo_ref[...],
                               preferred_element_type=jnp.float32)

    f32 = jnp.float32
    bf16 = jnp.bfloat16
    myoff = pl.multiple_of(8 * m, 8)

    stage_ref[0:SQ, :] = out_ref[0, :, :].astype(bf16)
    rs = []
    for d in range(1, N_DEV):
        p = (m + d) % N_DEV
        r = pltpu.make_async_remote_copy(
            src_ref=stage_ref.at[pl.ds(pl.multiple_of(8 * p, 8), 8), :],
            dst_ref=comm_ref.at[pl.ds(pl.multiple_of(_CR + 8 * m, 8), 8), :],
            send_sem=send_sems.at[_SEM_RS + d - 1],
            recv_sem=recv_sems.at[_SEM_RS + d - 1],
            device_id=(p,),
            device_id_type=MESH,
        )
        r.start()
        rs.append(r)
    comm_ref[pl.ds(pl.multiple_of(_CR + myoff, 8), 8), :] = (
        stage_ref[pl.ds(myoff, 8), :])
    for r in rs:
        r.wait_send()
    for r in rs:
        r.wait_recv()
    reduced = jnp.sum(
        comm_ref[_CR:_CR + SQ, :].astype(f32).reshape(N_DEV, 8, D), axis=0)
    out_ref[0, pl.ds(myoff, 8), :] = reduced

    stage_ref[0:8, :] = reduced.astype(bf16)
    ag = []
    for d in range(1, N_DEV):
        p = (m + d) % N_DEV
        r = pltpu.make_async_remote_copy(
            src_ref=stage_ref.at[0:8, :],
            dst_ref=comm_ref.at[pl.ds(pl.multiple_of(_CA + 8 * m, 8), 8), :],
            send_sem=send_sems.at[_SEM_AG + d - 1],
            recv_sem=recv_sems.at[_SEM_AG + d - 1],
            device_id=(p,),
            device_id_type=MESH,
        )
        r.start()
        ag.append(r)
    comm_ref[pl.ds(pl.multiple_of(_CA + myoff, 8), 8), :] = stage_ref[0:8, :]
    for r in ag:
        r.wait_send()
    for r in ag:
        r.wait_recv()
    out_ref[0, :, :] = comm_ref[_CA:_CA + SQ, :].astype(f32)


def kernel(x, Wq, Wo, K_ext, V_ext):
    return pl.pallas_call(
        _body,
        out_shape=jax.ShapeDtypeStruct((1, SQ, D), jnp.float32),
        in_specs=[
            pl.BlockSpec(memory_space=pltpu.VMEM),
            pl.BlockSpec(memory_space=pltpu.VMEM),
            pl.BlockSpec(memory_space=pltpu.VMEM),
            pl.BlockSpec(memory_space=pl.ANY),
            pl.BlockSpec(memory_space=pl.ANY),
        ],
        out_specs=pl.BlockSpec(memory_space=pltpu.VMEM),
        scratch_shapes=[
            pltpu.VMEM((SKV, KV_LOCAL, DH), jnp.float32),
            pltpu.VMEM((SKV, KV_LOCAL, DH), jnp.float32),
            pltpu.SemaphoreType.DMA((2,)),
            pltpu.VMEM((512, D), jnp.bfloat16),
            pltpu.VMEM((SQ, D), jnp.bfloat16),
            pltpu.SemaphoreType.DMA((N_SEMS,)),
            pltpu.SemaphoreType.DMA((N_SEMS,)),
        ],
        compiler_params=pltpu.CompilerParams(
            collective_id=0,
            vmem_limit_bytes=100 * 1024 * 1024),
    )(x, Wq, Wo, K_ext, V_ext)
